# Initial kernel scaffold; baseline (speedup 1.0000x reference)
import jax
import jax.numpy as jnp
from jax.experimental import pallas as pl


def kernel(xh_lig, xh_context, t, mask_lig, mask_context, params):
    raise NotImplementedError("write your pallas kernel here")


# speedup vs baseline: 69.8489x; 69.8489x over previous
import functools

import jax
import jax.numpy as jnp
from jax.experimental import pallas as pl
from jax.experimental.pallas import tpu as pltpu

_NDIM = 3
_HID = 64
_NLAYERS = 4
_SUB = 2
_INV_NF = 0.01
_NC = 1.0
_NL = 1024
_NCX = 2048
_TI = 64
_TJ = 128
_NTI = _NL // _TI
_NTJL = _NL // _TJ
_NTJC = _NCX // _TJ
_F32 = jnp.float32


def _silu(v):
    return v * jax.nn.sigmoid(v)


def _mm(a, b):
    return jax.lax.dot_general(a, b, (((1,), (0,)), ((), ())),
                               preferred_element_type=_F32)


def _lnorm(z):
    m = jnp.mean(z, axis=-1, keepdims=True)
    v = jnp.mean((z - m) ** 2, axis=-1, keepdims=True)
    return (z - m) / jnp.sqrt(v + 1e-5)


def _rep_i(v, k):
    return jnp.broadcast_to(v[:, None, :], (_TI, _TJ, k)).reshape(_TI * _TJ, k)


def _rep_j(v, k):
    return jnp.broadcast_to(v[None, :, :], (_TI, _TJ, k)).reshape(_TI * _TJ, k)


def _pack_net(p, t0):
    blocks = p['blocks']

    def st(f):
        return jnp.stack([jnp.stack([f(b['gcls'][s]) for s in range(_SUB)])
                          for b in blocks])

    def stc(f):
        return jnp.stack([f(b['coord']) for b in blocks])

    return {
        'Er': st(lambda g: g['e1']['w'][:_HID]),
        'Ec': st(lambda g: g['e1']['w'][_HID:2 * _HID]),
        'Ed': st(lambda g: g['e1']['w'][2 * _HID:]),
        'Eb': st(lambda g: g['e1']['b'][None]),
        'E2w': st(lambda g: g['e2']['w']),
        'E2b': st(lambda g: g['e2']['b'][None]),
        'N1h': st(lambda g: g['n1']['w'][:_HID]),
        'N1a': st(lambda g: g['n1']['w'][_HID:]),
        'N1b': st(lambda g: g['n1']['b'][None]),
        'N2w': st(lambda g: g['n2']['w']),
        'N2b': st(lambda g: g['n2']['b'][None]),
        'Cr': stc(lambda c: c['c1']['w'][:_HID]),
        'Cc': stc(lambda c: c['c1']['w'][_HID:2 * _HID]),
        'Cd': stc(lambda c: c['c1']['w'][2 * _HID:]),
        'Cb': stc(lambda c: c['c1']['b'][None]),
        'C2w': stc(lambda c: c['c2']['w']),
        'C2b': stc(lambda c: c['c2']['b'][None]),
        'C3w': stc(lambda c: c['c3']['w']),
        'C3b': stc(lambda c: c['c3']['b'][None]),
        'Wi': p['emb_in']['w'][:_HID],
        'bi': (p['emb_in']['b'] + t0 * p['emb_in']['w'][_HID])[None],
        'Wo': p['emb_out']['w'][:, :_HID],
        'bo': p['emb_out']['b'][None, :_HID],
    }


def _mega(treedef, n_in, *refs):
    mlS = refs[0]
    mcS = refs[1]
    ins = jax.tree.unflatten(treedef, refs[2:2 + n_in])
    out = refs[2 + n_in]
    (s_h, s_x, s_agg, s_xagg, s_hr, s_hc,
     s_hlemb, s_hkv, s_hres, s_xres) = refs[3 + n_in:]

    e = ins['enc']
    z = _silu(_mm(ins['hl0'][:], e['a0w'][:]) + e['a0b'][:])
    z = _mm(z, e['a1w'][:]) + e['a1b'][:]
    s_hlemb[:] = _lnorm(jnp.clip(z, -50.0, 50.0))
    z = _silu(_mm(ins['hp0'][:], e['c0w'][:]) + e['c0b'][:])
    z = _mm(z, e['c1w'][:]) + e['c1b'][:]
    s_hkv[:] = _lnorm(jnp.clip(z, -50.0, 50.0))

    def tile_loop(ntj, mkS, body_fn):
        def body(tid, carry):
            i = tid // ntj
            j = tid - i * ntj
            ib = pl.multiple_of(i * _TI, _TI)
            jb = pl.multiple_of(j * _TJ, _TJ)
            lo_i = mlS[ib]
            hi_i = mlS[ib + _TI - 1]
            lo_j = mkS[jb]
            hi_j = mkS[jb + _TJ - 1]

            @pl.when(jnp.logical_and(lo_i <= hi_j, lo_j <= hi_i))
            def _():
                body_fn(i, j, ib, jb)

            return carry

        jax.lax.fori_loop(0, _NTI * ntj, body, 0, unroll=False)

    def run_network(W, cross):
        ntj = _NTJC if cross else _NTJL
        nk = ntj * _TJ
        mkS = mcS if cross else mlS
        mkc = ins['mcc'] if cross else ins['mlc']
        hk = s_hkv if cross else s_h
        xk = ins['xp'] if cross else s_x

        s_h[:] = _mm(s_hlemb[:], W['Wi'][:]) + W['bi'][:]
        s_x[:] = ins['xl'][:]

        def edge_common(i, j, ib, jb):
            xi = s_x[pl.ds(ib, _TI), :]
            xj = xk[pl.ds(jb, _TJ), :]
            D = _rep_i(xi, _NDIM) - _rep_j(xj, _NDIM)
            d2 = jnp.sum(D * D, axis=1, keepdims=True)
            mq = ins['mlc'][pl.ds(ib, _TI), :]
            mk = mkc[pl.ds(jb, _TJ), :]
            eqm = _rep_i(mq, 1) == _rep_j(mk, 1)
            if not cross:
                gi = ins['gidx'][pl.ds(ib, _TI), :]
                gj = ins['gidx'][pl.ds(jb, _TJ), :]
                eqm = jnp.logical_and(
                    eqm, _rep_i(gi, 1) != _rep_j(gj, 1))
            return D, d2, eqm.astype(_F32)

        def gcl_pass(blk, s):
            ed = W['Ed'][blk, s]
            s_hr[:] = _mm(s_h[:], W['Er'][blk, s])
            s_hc[0:nk, :] = _mm(hk[:], W['Ec'][blk, s]) + W['Eb'][blk, s]
            s_agg[:] = jnp.zeros((_NL, _HID), _F32)
            e2w = W['E2w'][blk, s]
            e2b = W['E2b'][blk, s]

            def body(i, j, ib, jb):
                _, d2, w = edge_common(i, j, ib, jb)
                pre = (_rep_i(s_hr[pl.ds(ib, _TI), :], _HID)
                       + _rep_j(s_hc[pl.ds(jb, _TJ), :], _HID)
                       + d2 * ed)
                m = _silu(_mm(_silu(pre), e2w) + e2b) * w
                s_agg[pl.ds(ib, _TI), :] += jnp.sum(
                    m.reshape(_TI, _TJ, _HID), axis=1)

            tile_loop(ntj, mkS, body)
            h = s_h[:]
            agg = s_agg[:] * _INV_NF
            u = _silu(_mm(h, W['N1h'][blk, s]) + _mm(agg, W['N1a'][blk, s])
                      + W['N1b'][blk, s])
            s_h[:] = h + _mm(u, W['N2w'][blk, s]) + W['N2b'][blk, s]

        def coord_pass(blk):
            ed = W['Cd'][blk]
            s_hr[:] = _mm(s_h[:], W['Cr'][blk])
            s_hc[0:nk, :] = _mm(hk[:], W['Cc'][blk]) + W['Cb'][blk]
            s_xagg[:] = jnp.zeros((_NL, _NDIM), _F32)
            c2w = W['C2w'][blk]
            c2b = W['C2b'][blk]
            c3w = W['C3w'][blk]
            c3b = W['C3b'][blk]

            def body(i, j, ib, jb):
                D, d2, w = edge_common(i, j, ib, jb)
                pre = (_rep_i(s_hr[pl.ds(ib, _TI), :], _HID)
                       + _rep_j(s_hc[pl.ds(jb, _TJ), :], _HID)
                       + d2 * ed)
                a2 = _silu(_mm(_silu(pre), c2w) + c2b)
                phi = jnp.tanh(_mm(a2, c3w) + c3b)
                scale = phi * w / (jnp.sqrt(d2 + 1e-8) + _NC)
                s_xagg[pl.ds(ib, _TI), :] += jnp.sum(
                    (D * scale).reshape(_TI, _TJ, _NDIM), axis=1)

            tile_loop(ntj, mkS, body)
            s_x[:] = s_x[:] + s_xagg[:] * _INV_NF

        def blk_body(blk, carry):
            for s in range(_SUB):
                gcl_pass(blk, s)
            coord_pass(blk)
            return carry

        jax.lax.fori_loop(0, _NLAYERS, blk_body, 0, unroll=False)
        return _mm(s_h[:], W['Wo'][:]) + W['bo'][:]

    s_hres[:] = run_network(ins['egnn'], cross=False)
    s_xres[:] = s_x[:]
    hlp = run_network(ins['cross'], cross=True)

    xl = ins['xl'][:]
    vel = 0.6 * (s_xres[:] - xl) + 0.4 * (s_x[:] - xl)
    hfe = 0.6 * s_hres[:] + 0.4 * hlp
    d = ins['dec']
    z = _silu(_mm(hfe, d['d0w'][:]) + d['d0b'][:])
    hf = _mm(z, d['d1w'][:]) + d['d1b'][:]
    out[:] = jnp.concatenate([vel, hf], axis=1)


def kernel(xh_lig, xh_context, t, mask_lig, mask_context, params):
    kj = jax.random.key(42)
    x_l = xh_lig[:, :_NDIM] + 0.001 * jax.random.normal(
        jax.random.fold_in(kj, 1), (_NL, _NDIM), dtype=_F32)
    x_p = xh_context[:, :_NDIM] + 0.001 * jax.random.normal(
        jax.random.fold_in(kj, 2), (_NCX, _NDIM), dtype=_F32)
    t0 = t[0]
    bundle = {
        'hl0': xh_lig[:, _NDIM:],
        'hp0': xh_context[:, _NDIM:],
        'xl': x_l,
        'xp': x_p,
        'mlc': mask_lig.astype(_F32).reshape(_NL, 1),
        'mcc': mask_context.astype(_F32).reshape(_NCX, 1),
        'gidx': jnp.arange(_NL, dtype=_F32).reshape(_NL, 1),
        'enc': {
            'a0w': params['atom_enc'][0]['w'], 'a0b': params['atom_enc'][0]['b'][None],
            'a1w': params['atom_enc'][1]['w'], 'a1b': params['atom_enc'][1]['b'][None],
            'c0w': params['ctx_enc'][0]['w'], 'c0b': params['ctx_enc'][0]['b'][None],
            'c1w': params['ctx_enc'][1]['w'], 'c1b': params['ctx_enc'][1]['b'][None],
        },
        'dec': {
            'd0w': params['atom_dec'][0]['w'], 'd0b': params['atom_dec'][0]['b'][None],
            'd1w': params['atom_dec'][1]['w'], 'd1b': params['atom_dec'][1]['b'][None],
        },
        'egnn': _pack_net(params['egnn'], t0),
        'cross': _pack_net(params['cross'], t0),
    }
    leaves, treedef = jax.tree.flatten(bundle)
    mlS = mask_lig.astype(jnp.int32)
    mcS = mask_context.astype(jnp.int32)
    fn = pl.pallas_call(
        functools.partial(_mega, treedef, len(leaves)),
        out_shape=jax.ShapeDtypeStruct((_NL, _NDIM + _HID), _F32),
        in_specs=([pl.BlockSpec(memory_space=pltpu.SMEM)] * 2
                  + [pl.BlockSpec(memory_space=pltpu.VMEM)] * len(leaves)),
        out_specs=pl.BlockSpec(memory_space=pltpu.VMEM),
        scratch_shapes=[
            pltpu.VMEM((_NL, _HID), _F32),
            pltpu.VMEM((_NL, _NDIM), _F32),
            pltpu.VMEM((_NL, _HID), _F32),
            pltpu.VMEM((_NL, _NDIM), _F32),
            pltpu.VMEM((_NL, _HID), _F32),
            pltpu.VMEM((_NCX, _HID), _F32),
            pltpu.VMEM((_NL, _HID), _F32),
            pltpu.VMEM((_NCX, _HID), _F32),
            pltpu.VMEM((_NL, _HID), _F32),
            pltpu.VMEM((_NL, _NDIM), _F32),
        ],
    )
    return fn(mlS, mcS, *leaves)
